# baseline (speedup 1.0000x reference)
import functools

import jax
import jax.numpy as jnp
from jax import lax
from jax.experimental import pallas as pl
from jax.experimental.pallas import tpu as pltpu
from jax.experimental.pallas import tpu_sc as plsc

D = 128
DP = 144
NEG_SLOPE = 0.2
EPS = 1e-16
NC, NS, NW = 2, 16, 32
K = 80


BLK = 2048


def _normalize(agg_ref, den0_ref, den1_ref):
    a = agg_ref[0] + agg_ref[1]
    dd = den0_ref[...] + den1_ref[...]
    return a / (dd[:, None] + EPS)


def _tc_head_tail(x, w_ref, asrc_ref, adst_ref, h_ref, ssrc_ref, sdst_ref):
    w = w_ref[...]
    vsrc = jnp.dot(w, asrc_ref[0, :][:, None],
                   preferred_element_type=jnp.float32)
    vdst = jnp.dot(w, adst_ref[0, :][:, None],
                   preferred_element_type=jnp.float32)
    h_ref[...] = jnp.dot(x, w, preferred_element_type=jnp.float32)
    ssrc_ref[...] = jnp.sum(x * vsrc[:, 0][None, :], axis=1)
    sdst_ref[...] = jnp.sum(x * vdst[:, 0][None, :], axis=1)


def _tc_head0_body(x_ref, w_ref, asrc_ref, adst_ref, *outs):
    _tc_head_tail(x_ref[...], w_ref, asrc_ref, adst_ref, *outs)


def _tc_headn_body(agg_ref, den0_ref, den1_ref, w_ref, asrc_ref, adst_ref,
                   *outs):
    _tc_head_tail(_normalize(agg_ref, den0_ref, den1_ref), w_ref, asrc_ref,
                  adst_ref, *outs)


def _wspecs():
    return [
        pl.BlockSpec((D, D), lambda i: (0, 0)),
        pl.BlockSpec((1, D), lambda i: (0, 0)),
        pl.BlockSpec((1, D), lambda i: (0, 0)),
    ]


def _den_specs(n):
    nb = n // BLK
    return [
        pl.BlockSpec((BLK,), lambda i: (i,)),
        pl.BlockSpec((BLK,), lambda i, _nb=nb: (_nb + i,)),
    ]


def _tc_head(x_or_agg, den, w, asrc, adst, n, is_first):
    grid = (n // BLK,)
    out_shape = [
        jax.ShapeDtypeStruct((n, D), jnp.float32),
        jax.ShapeDtypeStruct((n,), jnp.float32),
        jax.ShapeDtypeStruct((n,), jnp.float32),
    ]
    out_specs = [
        pl.BlockSpec((BLK, D), lambda i: (i, 0)),
        pl.BlockSpec((BLK,), lambda i: (i,)),
        pl.BlockSpec((BLK,), lambda i: (i,)),
    ]
    if is_first:
        return pl.pallas_call(
            _tc_head0_body, grid=grid,
            in_specs=[pl.BlockSpec((BLK, D), lambda i: (i, 0))] + _wspecs(),
            out_specs=out_specs, out_shape=out_shape)(x_or_agg, w, asrc, adst)
    return pl.pallas_call(
        _tc_headn_body, grid=grid,
        in_specs=([pl.BlockSpec((NC, BLK, D), lambda i: (0, i, 0))]
                  + _den_specs(n) + _wspecs()),
        out_specs=out_specs, out_shape=out_shape)(
            x_or_agg, den, den, w, asrc, adst)


def _tc_final_body(agg_ref, den0_ref, den1_ref, out_ref):
    x = _normalize(agg_ref, den0_ref, den1_ref)
    out_ref[...] = jnp.where(x > 0, x, jnp.exp(jnp.minimum(x, 0.0)) - 1.0)


def _tc_final(agg, den, n):
    return pl.pallas_call(
        _tc_final_body,
        grid=(n // BLK,),
        in_specs=([pl.BlockSpec((NC, BLK, D), lambda i: (0, i, 0))]
                  + _den_specs(n)),
        out_specs=pl.BlockSpec((BLK, D), lambda i: (i, 0)),
        out_shape=jax.ShapeDtypeStruct((n, D), jnp.float32),
    )(agg, den, den)


def _sc_layout(e):
    nchunk = -(-e // (8 * K)) * 8
    per = -(-nchunk // NW)
    per = -(-per // 8) * 8
    return per, nchunk, nchunk * K


def _make_sc_edge_pass(n, e):
    per, nchunk, _ = _sc_layout(e)
    na = n + 8
    blk = 40
    nblk = n // blk
    bper, brem = nblk // NS, nblk % NS
    mesh = plsc.VectorSubcoreMesh(core_axis_name="c", subcore_axis_name="s")

    @functools.partial(
        pl.kernel,
        out_type=(jax.ShapeDtypeStruct((NC, n, D), jnp.float32),
                  jax.ShapeDtypeStruct((NC * n,), jnp.float32)),
        mesh=mesh,
        scratch_types=[
            pltpu.VMEM((8, 2, K), jnp.int32),
            pltpu.VMEM((2, K, D), jnp.float32),
            pltpu.VMEM((2, K), jnp.float32),
            pltpu.VMEM((n,), jnp.float32),
            pltpu.VMEM((n,), jnp.float32),
            pltpu.VMEM_SHARED((na, D), jnp.float32),
            pltpu.VMEM_SHARED((na,), jnp.float32),
            pltpu.SemaphoreType.DMA,
            pltpu.SemaphoreType.DMA,
            pltpu.SemaphoreType.DMA,
            pltpu.SemaphoreType.DMA,
            pltpu.SemaphoreType.DMA,
            pltpu.SemaphoreType.DMA,
            pltpu.SemaphoreType.DMA,
            pltpu.SemaphoreType.DMA,
        ],
        compiler_params=pltpu.CompilerParams(needs_layout_passes=False,
                                             use_tc_tiling_on_sc=False),
    )
    def sc_edge_pass(h_hbm, ssrc_hbm, sdst_hbm, sd_hbm, out_hbm, den_hbm,
                     sdix, rows, exb, ssl, sdl, agg, den,
                     isem0, isem1, isem2, isem3, gsem0, gsem1,
                     ssem0, ssem1):
        c = lax.axis_index("c")
        s = lax.axis_index("s")
        w = s * NC + c
        wc = w * per
        isems = (isem0, isem1, isem2, isem3)
        gsems = (gsem0, gsem1)
        ssems = (ssem0, ssem1)

        def _issue_idx(j, slot):
            pltpu.async_copy(sd_hbm.at[wc + j], sdix.at[slot],
                             isems[slot % 4])

        def _wait_idx(slot):
            pltpu.make_async_copy(sd_hbm.at[0], sdix.at[slot],
                                  isems[slot % 4]).wait()

        def _issue_gather(slot, p):
            pltpu.async_copy(h_hbm.at[sdix.at[slot, 0]], rows.at[p],
                             gsems[p])

        def _wait_gather(p):
            pltpu.make_async_copy(h_hbm.at[pl.ds(0, K)], rows.at[p],
                                  gsems[p]).wait()

        def _issue_scatter(slot, p):
            pltpu.async_copy(rows.at[p], agg.at[sdix.at[slot, 1]], ssems[p],
                             add=True)
            pltpu.async_copy(exb.at[p], den.at[sdix.at[slot, 1]], ssems[p],
                             add=True)

        def _wait_scatter(p):
            pltpu.make_async_copy(h_hbm.at[pl.ds(0, K)], rows.at[p],
                                  ssems[p]).wait()
            pltpu.make_async_copy(ssrc_hbm.at[pl.ds(0, K)], exb.at[p],
                                  ssems[p]).wait()

        def _zero_rows(k, _):
            for q in range(D // 16):
                rows[0, k, pl.ds(q * 16, 16)] = jnp.zeros((16,), jnp.float32)
            return 0
        lax.fori_loop(0, blk, _zero_rows, 0)
        nb = bper + jnp.where(s < brem, 1, 0)

        def _zero_blk(i, _):
            off = (s + NS * i) * blk
            pltpu.sync_copy(rows.at[0, pl.ds(0, blk)], agg.at[pl.ds(off, blk)])
            pltpu.sync_copy(rows.at[0, 0, pl.ds(0, blk)],
                            den.at[pl.ds(off, blk)])
            return 0
        lax.fori_loop(0, nb, _zero_blk, 0)

        @pl.when(s == NS - 1)
        def _zero_trash():
            pltpu.sync_copy(rows.at[0, pl.ds(0, 8)], agg.at[pl.ds(n, 8)])
            pltpu.sync_copy(rows.at[0, 0, pl.ds(0, 8)], den.at[pl.ds(n, 8)])

        pltpu.sync_copy(ssrc_hbm, ssl)
        pltpu.sync_copy(sdst_hbm, sdl)
        plsc.subcore_barrier()

        nreal = jnp.clip(nchunk - wc, 0, per)

        def _do_chunk(slot, p):
            for g in range(K // 16):
                sv = sdix[slot, 0, pl.ds(g * 16, 16)]
                dv = sdix[slot, 1, pl.ds(g * 16, 16)]
                t = plsc.load_gather(ssl, [sv]) + plsc.load_gather(sdl, [dv])
                t = jnp.maximum(t, NEG_SLOPE * t)
                exb[p, pl.ds(g * 16, 16)] = jnp.exp(t)
            _wait_gather(p)

            def mul_body(g, _):
                ex_v = exb[p, pl.ds(g * 16, 16)]
                for i in range(16):
                    sc = ex_v[i]
                    k = g * 16 + i
                    for q in range(D // 16):
                        rows[p, k, pl.ds(q * 16, 16)] = (
                            rows[p, k, pl.ds(q * 16, 16)] * sc)
                return 0
            lax.fori_loop(0, K // 16, mul_body, 0)
            _issue_scatter(slot, p)

        @pl.when(nreal > 0)
        def _edge_pipeline():
            for j in range(4):
                _issue_idx(j, j)
            _wait_idx(0)
            _issue_gather(0, 0)
            _wait_idx(1)
            _issue_gather(1, 1)

            def octet_body(v, _):
                base = 8 * v
                for q in range(4):
                    _do_chunk(2 * q, 0)
                    _do_chunk(2 * q + 1, 1)
                    cg0 = base + 2 * q + 2
                    cg1 = base + 2 * q + 3

                    @pl.when(cg0 < nreal)
                    def _pf0():
                        _wait_idx((2 * q + 2) % 8)
                        _wait_scatter(0)
                        _issue_gather((2 * q + 2) % 8, 0)

                    @pl.when(cg1 < nreal)
                    def _pf1():
                        _wait_idx((2 * q + 3) % 8)
                        _wait_scatter(1)
                        _issue_gather((2 * q + 3) % 8, 1)

                    @pl.when(base + 2 * q + 4 < nreal)
                    def _pi0():
                        _issue_idx(base + 2 * q + 4, (2 * q + 4) % 8)

                    @pl.when(base + 2 * q + 5 < nreal)
                    def _pi1():
                        _issue_idx(base + 2 * q + 5, (2 * q + 5) % 8)
                return 0
            lax.fori_loop(0, nreal // 8, octet_body, 0)
            _wait_scatter(0)
            _wait_scatter(1)

        plsc.subcore_barrier()

        def _dump_blk(i, _):
            off = (s + NS * i) * blk
            pltpu.sync_copy(agg.at[pl.ds(off, blk)],
                            out_hbm.at[c, pl.ds(off, blk)])
            pltpu.sync_copy(den.at[pl.ds(off, blk)],
                            den_hbm.at[pl.ds(c * n + off, blk)])
            return 0
        lax.fori_loop(0, nb, _dump_blk, 0)

    return sc_edge_pass


def kernel(x, edge_index, W, a_src, a_dst):
    n = x.shape[0]
    np_ = -(-n // BLK) * BLK
    e = edge_index.shape[1]
    per, nchunk, e_pad = _sc_layout(e)
    src = jnp.pad(edge_index[0], (0, e_pad - e)).reshape(-1, K)
    trash = np_ + (jnp.arange(e_pad - e, dtype=jnp.int32) % 8)
    dst = jnp.concatenate([edge_index[1], trash]).reshape(-1, K)
    sd = jnp.stack([src, dst], axis=1)
    sc_pass = _make_sc_edge_pass(np_, e)

    agg, den = jnp.pad(x, ((0, np_ - n), (0, 0))), None
    for i in range(W.shape[0]):
        h, ssrc, sdst = _tc_head(
            agg, den, W[i], a_src[i][None, :], a_dst[i][None, :], np_,
            is_first=(i == 0))
        agg, den = sc_pass(h, ssrc, sdst, sd)
    return _tc_final(agg, den, np_)[:n]


# speedup vs baseline: 1.0169x; 1.0169x over previous
import functools

import jax
import jax.numpy as jnp
from jax import lax
from jax.experimental import pallas as pl
from jax.experimental.pallas import tpu as pltpu
from jax.experimental.pallas import tpu_sc as plsc

D = 128
DP = 144
NEG_SLOPE = 0.2
EPS = 1e-16
NC, NS, NW = 2, 16, 32
K = 80


def _normalize(agg_ref, den_ref, n):
    a = agg_ref[0] + agg_ref[1]
    d = den_ref[...]
    dd = d[:n] + d[n:]
    return a / (dd[:, None] + EPS)


def _tc_head_tail(x, w_ref, asrc_ref, adst_ref, h_ref, ssrc_ref, sdst_ref):
    w = w_ref[...]
    vsrc = jnp.dot(w, asrc_ref[0, :][:, None],
                   preferred_element_type=jnp.float32)
    vdst = jnp.dot(w, adst_ref[0, :][:, None],
                   preferred_element_type=jnp.float32)
    h_ref[...] = jnp.dot(x, w, preferred_element_type=jnp.float32)
    ssrc_ref[...] = jnp.sum(x * vsrc[:, 0][None, :], axis=1)
    sdst_ref[...] = jnp.sum(x * vdst[:, 0][None, :], axis=1)


def _tc_head0_body(x_ref, w_ref, asrc_ref, adst_ref, *outs):
    _tc_head_tail(x_ref[...], w_ref, asrc_ref, adst_ref, *outs)


def _tc_headn_body(n, agg_ref, den_ref, w_ref, asrc_ref, adst_ref, *outs):
    _tc_head_tail(_normalize(agg_ref, den_ref, n), w_ref, asrc_ref,
                  adst_ref, *outs)


def _tc_head(x_or_agg, den, w, asrc, adst, n, is_first):
    out_shape = [
        jax.ShapeDtypeStruct((n, D), jnp.float32),
        jax.ShapeDtypeStruct((n,), jnp.float32),
        jax.ShapeDtypeStruct((n,), jnp.float32),
    ]
    if is_first:
        return pl.pallas_call(_tc_head0_body, out_shape=out_shape)(
            x_or_agg, w, asrc, adst)
    return pl.pallas_call(
        functools.partial(_tc_headn_body, n), out_shape=out_shape)(
            x_or_agg, den, w, asrc, adst)


def _tc_final_body(n, agg_ref, den_ref, out_ref):
    x = _normalize(agg_ref, den_ref, n)
    out_ref[...] = jnp.where(x > 0, x, jnp.exp(jnp.minimum(x, 0.0)) - 1.0)


def _tc_final(agg, den, n):
    return pl.pallas_call(
        functools.partial(_tc_final_body, n),
        out_shape=jax.ShapeDtypeStruct((n, D), jnp.float32),
    )(agg, den)


def _sc_layout(e):
    nchunk = -(-e // (8 * K)) * 8
    per = -(-nchunk // NW)
    per = -(-per // 8) * 8
    return per, nchunk, nchunk * K


def _make_sc_edge_pass(n, e):
    per, nchunk, _ = _sc_layout(e)
    na = n + 8
    blk = 40
    nblk = n // blk
    bper, brem = nblk // NS, nblk % NS
    mesh = plsc.VectorSubcoreMesh(core_axis_name="c", subcore_axis_name="s")

    @functools.partial(
        pl.kernel,
        out_type=(jax.ShapeDtypeStruct((NC, n, D), jnp.float32),
                  jax.ShapeDtypeStruct((NC * n,), jnp.float32)),
        mesh=mesh,
        scratch_types=[
            pltpu.VMEM((8, 2, K), jnp.int32),
            pltpu.VMEM((2, K, D), jnp.float32),
            pltpu.VMEM((2, K), jnp.float32),
            pltpu.VMEM((n,), jnp.float32),
            pltpu.VMEM((n,), jnp.float32),
            pltpu.VMEM_SHARED((na, D), jnp.float32),
            pltpu.VMEM_SHARED((na,), jnp.float32),
            pltpu.SemaphoreType.DMA,
            pltpu.SemaphoreType.DMA,
            pltpu.SemaphoreType.DMA,
            pltpu.SemaphoreType.DMA,
            pltpu.SemaphoreType.DMA,
            pltpu.SemaphoreType.DMA,
            pltpu.SemaphoreType.DMA,
            pltpu.SemaphoreType.DMA,
        ],
        compiler_params=pltpu.CompilerParams(needs_layout_passes=False,
                                             use_tc_tiling_on_sc=False),
    )
    def sc_edge_pass(h_hbm, ssrc_hbm, sdst_hbm, sd_hbm, out_hbm, den_hbm,
                     sdix, rows, exb, ssl, sdl, agg, den,
                     isem0, isem1, isem2, isem3, gsem0, gsem1,
                     ssem0, ssem1):
        c = lax.axis_index("c")
        s = lax.axis_index("s")
        w = s * NC + c
        wc = w * per
        isems = (isem0, isem1, isem2, isem3)
        gsems = (gsem0, gsem1)
        ssems = (ssem0, ssem1)

        def _issue_idx(j, slot):
            pltpu.async_copy(sd_hbm.at[wc + j], sdix.at[slot],
                             isems[slot % 4])

        def _wait_idx(slot):
            pltpu.make_async_copy(sd_hbm.at[0], sdix.at[slot],
                                  isems[slot % 4]).wait()

        def _issue_gather(slot, p):
            pltpu.async_copy(h_hbm.at[sdix.at[slot, 0]], rows.at[p],
                             gsems[p])

        def _wait_gather(p):
            pltpu.make_async_copy(h_hbm.at[pl.ds(0, K)], rows.at[p],
                                  gsems[p]).wait()

        def _issue_scatter(slot, p):
            pltpu.async_copy(rows.at[p], agg.at[sdix.at[slot, 1]], ssems[p],
                             add=True)
            pltpu.async_copy(exb.at[p], den.at[sdix.at[slot, 1]], ssems[p],
                             add=True)

        def _wait_scatter(p):
            pltpu.make_async_copy(h_hbm.at[pl.ds(0, K)], rows.at[p],
                                  ssems[p]).wait()
            pltpu.make_async_copy(ssrc_hbm.at[pl.ds(0, K)], exb.at[p],
                                  ssems[p]).wait()

        def _zero_rows(k, _):
            for q in range(D // 16):
                rows[0, k, pl.ds(q * 16, 16)] = jnp.zeros((16,), jnp.float32)
            return 0
        lax.fori_loop(0, blk, _zero_rows, 0)
        nb = bper + jnp.where(s < brem, 1, 0)

        def _zero_blk(i, _):
            off = (s + NS * i) * blk
            pltpu.sync_copy(rows.at[0, pl.ds(0, blk)], agg.at[pl.ds(off, blk)])
            pltpu.sync_copy(rows.at[0, 0, pl.ds(0, blk)],
                            den.at[pl.ds(off, blk)])
            return 0
        lax.fori_loop(0, nb, _zero_blk, 0)

        @pl.when(s == NS - 1)
        def _zero_trash():
            pltpu.sync_copy(rows.at[0, pl.ds(0, 8)], agg.at[pl.ds(n, 8)])
            pltpu.sync_copy(rows.at[0, 0, pl.ds(0, 8)], den.at[pl.ds(n, 8)])

        pltpu.sync_copy(ssrc_hbm, ssl)
        pltpu.sync_copy(sdst_hbm, sdl)
        plsc.subcore_barrier()

        nreal = jnp.clip(nchunk - wc, 0, per)

        def _do_chunk(slot, p):
            for g in range(K // 16):
                sv = sdix[slot, 0, pl.ds(g * 16, 16)]
                dv = sdix[slot, 1, pl.ds(g * 16, 16)]
                t = plsc.load_gather(ssl, [sv]) + plsc.load_gather(sdl, [dv])
                t = jnp.maximum(t, NEG_SLOPE * t)
                exb[p, pl.ds(g * 16, 16)] = jnp.exp(t)
            _wait_gather(p)

            def mul_body(g, _):
                ex_v = exb[p, pl.ds(g * 16, 16)]
                for i in range(16):
                    sc = ex_v[i]
                    k = g * 16 + i
                    for q in range(D // 16):
                        rows[p, k, pl.ds(q * 16, 16)] = (
                            rows[p, k, pl.ds(q * 16, 16)] * sc)
                return 0
            lax.fori_loop(0, K // 16, mul_body, 0)
            _issue_scatter(slot, p)

        @pl.when(nreal > 0)
        def _edge_pipeline():
            for j in range(4):
                _issue_idx(j, j)
            _wait_idx(0)
            _issue_gather(0, 0)
            _wait_idx(1)
            _issue_gather(1, 1)

            def octet_body(v, _):
                base = 8 * v
                for q in range(4):
                    _do_chunk(2 * q, 0)
                    _do_chunk(2 * q + 1, 1)
                    cg0 = base + 2 * q + 2
                    cg1 = base + 2 * q + 3

                    @pl.when(cg0 < nreal)
                    def _pf0():
                        _wait_idx((2 * q + 2) % 8)
                        _wait_scatter(0)
                        _issue_gather((2 * q + 2) % 8, 0)

                    @pl.when(cg1 < nreal)
                    def _pf1():
                        _wait_idx((2 * q + 3) % 8)
                        _wait_scatter(1)
                        _issue_gather((2 * q + 3) % 8, 1)

                    @pl.when(base + 2 * q + 4 < nreal)
                    def _pi0():
                        _issue_idx(base + 2 * q + 4, (2 * q + 4) % 8)

                    @pl.when(base + 2 * q + 5 < nreal)
                    def _pi1():
                        _issue_idx(base + 2 * q + 5, (2 * q + 5) % 8)
                return 0
            lax.fori_loop(0, nreal // 8, octet_body, 0)
            _wait_scatter(0)
            _wait_scatter(1)

        plsc.subcore_barrier()

        def _dump_blk(i, _):
            off = (s + NS * i) * blk
            pltpu.sync_copy(agg.at[pl.ds(off, blk)],
                            out_hbm.at[c, pl.ds(off, blk)])
            pltpu.sync_copy(den.at[pl.ds(off, blk)],
                            den_hbm.at[pl.ds(c * n + off, blk)])
            return 0
        lax.fori_loop(0, nb, _dump_blk, 0)

    return sc_edge_pass


def kernel(x, edge_index, W, a_src, a_dst):
    n = x.shape[0]
    e = edge_index.shape[1]
    per, nchunk, e_pad = _sc_layout(e)
    src = jnp.pad(edge_index[0], (0, e_pad - e)).reshape(-1, K)
    trash = n + (jnp.arange(e_pad - e, dtype=jnp.int32) % 8)
    dst = jnp.concatenate([edge_index[1], trash]).reshape(-1, K)
    sd = jnp.stack([src, dst], axis=1)
    sc_pass = _make_sc_edge_pass(n, e)

    agg, den = x, None
    for i in range(W.shape[0]):
        h, ssrc, sdst = _tc_head(
            agg, den, W[i], a_src[i][None, :], a_dst[i][None, :], n,
            is_first=(i == 0))
        agg, den = sc_pass(h, ssrc, sdst, sd)
    return _tc_final(agg, den, n)
